# Initial kernel scaffold; baseline (speedup 1.0000x reference)
import jax
import jax.numpy as jnp
from jax.experimental import pallas as pl


def kernel(x, edge_index, batch, W1, b1, Wh, bh, W2, b2, Wp1, bp1, Wp2, bp2):
    raise NotImplementedError("write your pallas kernel here")


# speedup vs baseline: 8.3293x; 8.3293x over previous
import functools

import jax
import jax.numpy as jnp
from jax import lax
from jax.experimental import pallas as pl
from jax.experimental.pallas import tpu as pltpu
from jax.experimental.pallas import tpu_sc as plsc

_N = 10000
_E = 320000
_C = 128
_G = 16
_NC, _NS = 2, 16
_NW = _NC * _NS
_EPW = _E // _NW
_K = 80
_NCH = _EPW // _K
_NP = 10240
_RPS = _NP // _NS
_DW = 128

_R = 1000
_GRID = _N // _R

_mesh = plsc.VectorSubcoreMesh(core_axis_name="c", subcore_axis_name="s",
                               num_cores=_NC, num_subcores=_NS)


@functools.partial(
    pl.kernel,
    out_type=jax.ShapeDtypeStruct((_NC, _NP, _C), jnp.float32),
    mesh=_mesh,
    scratch_types=[
        pltpu.VMEM((_NCH, _K), jnp.int32),
        pltpu.VMEM((_NCH, _K), jnp.int32),
        pltpu.VMEM((_K, _C), jnp.float32),
        pltpu.VMEM_SHARED((_NP, _C), jnp.float32),
        pltpu.SemaphoreType.DMA,
    ],
)
def _sc_scatter(u_hbm, gidx_hbm, sidx_hbm, zeros_hbm, out_hbm,
                gi_v, si_v, rows_v, acc_sh, sem):
    c = lax.axis_index("c")
    s = lax.axis_index("s")
    w = c * _NS + s
    pltpu.sync_copy(zeros_hbm, acc_sh.at[pl.ds(s * _RPS, _RPS)])
    pltpu.sync_copy(gidx_hbm.at[w], gi_v)
    pltpu.sync_copy(sidx_hbm.at[w], si_v)
    plsc.subcore_barrier()

    def chunk(j, carry):
        pltpu.async_copy(u_hbm.at[gi_v.at[j]], rows_v, sem).wait()
        pltpu.sync_copy(rows_v, acc_sh.at[si_v.at[j]], add=True)
        return carry

    lax.fori_loop(0, _NCH, chunk, 0)
    plsc.subcore_barrier()
    pltpu.sync_copy(acc_sh.at[pl.ds(s * _RPS, _RPS)],
                    out_hbm.at[c, pl.ds(s * _RPS, _RPS)])


@functools.partial(
    pl.kernel,
    out_type=jax.ShapeDtypeStruct((_NC, _NP, _DW), jnp.float32),
    mesh=_mesh,
    scratch_types=[
        pltpu.VMEM((_NCH, _K), jnp.int32),
        pltpu.VMEM((_K, _DW), jnp.float32),
        pltpu.VMEM_SHARED((_NP, _DW), jnp.float32),
    ],
)
def _sc_degree(src_hbm, ones_hbm, zeros_hbm, out_hbm, si_v, ones_v, acc_sh):
    c = lax.axis_index("c")
    s = lax.axis_index("s")
    w = c * _NS + s
    pltpu.sync_copy(zeros_hbm, acc_sh.at[pl.ds(s * _RPS, _RPS)])
    pltpu.sync_copy(src_hbm.at[w], si_v)
    pltpu.sync_copy(ones_hbm, ones_v)
    plsc.subcore_barrier()

    def chunk(j, carry):
        pltpu.sync_copy(ones_v, acc_sh.at[si_v.at[j]], add=True)
        return carry

    lax.fori_loop(0, _NCH, chunk, 0)
    plsc.subcore_barrier()
    pltpu.sync_copy(acc_sh.at[pl.ds(s * _RPS, _RPS)],
                    out_hbm.at[c, pl.ds(s * _RPS, _RPS)])


_HI = lax.Precision.HIGHEST


def _dot(a, b):
    return jnp.dot(a, b, preferred_element_type=jnp.float32, precision=_HI)


def _tc_prologue_body(degp_ref, x_ref, dis_ref, u0_ref):
    deg = degp_ref[0, :, 0:1] + degp_ref[1, :, 0:1]
    dis = jnp.where(deg > 0.0, lax.rsqrt(jnp.maximum(deg, 1e-30)), 0.0)
    dis_ref[...] = jnp.broadcast_to(dis, (_R, _DW))
    u0_ref[...] = dis * x_ref[...]


_tc_prologue = pl.pallas_call(
    _tc_prologue_body,
    grid=(_GRID,),
    in_specs=[
        pl.BlockSpec((_NC, _R, _DW), lambda i: (0, i, 0)),
        pl.BlockSpec((_R, _C), lambda i: (i, 0)),
    ],
    out_specs=[
        pl.BlockSpec((_R, _DW), lambda i: (i, 0)),
        pl.BlockSpec((_R, _C), lambda i: (i, 0)),
    ],
    out_shape=[
        jax.ShapeDtypeStruct((_N, _DW), jnp.float32),
        jax.ShapeDtypeStruct((_N, _C), jnp.float32),
    ],
)


def _tc_stage_a_body(zp_ref, h_ref, dis_ref, w0_ref, w1_ref, b_ref,
                     part_ref, u1_ref):
    d = dis_ref[:, 0:1]
    z = zp_ref[0] + zp_ref[1]
    tx1 = -d * z
    part_ref[...] = _dot(h_ref[...], w0_ref[...]) + _dot(tx1, w1_ref[...]) \
        + b_ref[...]
    u1_ref[...] = d * tx1


_tc_stage_a = pl.pallas_call(
    _tc_stage_a_body,
    grid=(_GRID,),
    in_specs=[
        pl.BlockSpec((_NC, _R, _C), lambda i: (0, i, 0)),
        pl.BlockSpec((_R, _C), lambda i: (i, 0)),
        pl.BlockSpec((_R, _DW), lambda i: (i, 0)),
        pl.BlockSpec((_C, _C), lambda i: (0, 0)),
        pl.BlockSpec((_C, _C), lambda i: (0, 0)),
        pl.BlockSpec((1, _C), lambda i: (0, 0)),
    ],
    out_specs=[
        pl.BlockSpec((_R, _C), lambda i: (i, 0)),
        pl.BlockSpec((_R, _C), lambda i: (i, 0)),
    ],
    out_shape=[
        jax.ShapeDtypeStruct((_N, _C), jnp.float32),
        jax.ShapeDtypeStruct((_N, _C), jnp.float32),
    ],
)


def _tc_stage_b_body(zp_ref, part_ref, h_ref, dis_ref, w2_ref,
                     out_ref, un_ref, *, act):
    d = dis_ref[:, 0:1]
    z = zp_ref[0] + zp_ref[1]
    tx2 = -2.0 * d * z - h_ref[...]
    o = part_ref[...] + _dot(tx2, w2_ref[...])
    if act:
        o = o * lax.logistic(o)
    out_ref[...] = o
    un_ref[...] = d * o


def _make_stage_b(act):
    return pl.pallas_call(
        functools.partial(_tc_stage_b_body, act=act),
        grid=(_GRID,),
        in_specs=[
            pl.BlockSpec((_NC, _R, _C), lambda i: (0, i, 0)),
            pl.BlockSpec((_R, _C), lambda i: (i, 0)),
            pl.BlockSpec((_R, _C), lambda i: (i, 0)),
            pl.BlockSpec((_R, _DW), lambda i: (i, 0)),
            pl.BlockSpec((_C, _C), lambda i: (0, 0)),
        ],
        out_specs=[
            pl.BlockSpec((_R, _C), lambda i: (i, 0)),
            pl.BlockSpec((_R, _C), lambda i: (i, 0)),
        ],
        out_shape=[
            jax.ShapeDtypeStruct((_N, _C), jnp.float32),
            jax.ShapeDtypeStruct((_N, _C), jnp.float32),
        ],
    )


_tc_stage_b_act = _make_stage_b(True)
_tc_stage_b_lin = _make_stage_b(False)


def _tc_pool_body(b_ref, h_ref, wp1_ref, bp1_ref, wp2_ref, bp2_ref,
                  out_ref, acc_ref):
    i = pl.program_id(0)

    @pl.when(i == 0)
    def _zero():
        acc_ref[...] = jnp.zeros_like(acc_ref)

    onehot = (b_ref[...] == lax.broadcasted_iota(jnp.int32, (_R, _G), 1)
              ).astype(jnp.float32)
    acc_ref[...] += lax.dot_general(
        onehot, h_ref[...], (((0,), (0,)), ((), ())),
        preferred_element_type=jnp.float32, precision=_HI)

    @pl.when(i == pl.num_programs(0) - 1)
    def _head():
        p = acc_ref[...]
        o = jnp.maximum(_dot(p, wp1_ref[...]) + bp1_ref[...], 0.0)
        out_ref[...] = _dot(o, wp2_ref[...]) + bp2_ref[...]


_tc_pool = pl.pallas_call(
    _tc_pool_body,
    grid=(_GRID,),
    in_specs=[
        pl.BlockSpec((_R, 1), lambda i: (i, 0)),
        pl.BlockSpec((_R, _C), lambda i: (i, 0)),
        pl.BlockSpec((_C, 32), lambda i: (0, 0)),
        pl.BlockSpec((1, 32), lambda i: (0, 0)),
        pl.BlockSpec((32, _G), lambda i: (0, 0)),
        pl.BlockSpec((1, _G), lambda i: (0, 0)),
    ],
    out_specs=pl.BlockSpec((_G, _G), lambda i: (0, 0)),
    out_shape=jax.ShapeDtypeStruct((_G, _G), jnp.float32),
    scratch_shapes=[pltpu.VMEM((_G, _C), jnp.float32)],
)


def kernel(x, edge_index, batch, W1, b1, Wh, bh, W2, b2, Wp1, bp1, Wp2, bp2):
    src = edge_index[0].reshape(_NW, _NCH, _K)
    dst = edge_index[1].reshape(_NW, _NCH, _K)
    zeros_c = jnp.zeros((_RPS, _C), jnp.float32)
    zeros_d = jnp.zeros((_RPS, _DW), jnp.float32)
    ones_d = jnp.ones((_K, _DW), jnp.float32)
    batch2 = batch.reshape(_N, 1)

    degp = _sc_degree(src, ones_d, zeros_d)
    dis8, u = _tc_prologue(degp, x)

    layers = [
        (W1[0], W1[1], W1[2], b1, False),
        (Wh[0, 0], Wh[0, 1], Wh[0, 2], bh[0], True),
        (Wh[1, 0], Wh[1, 1], Wh[1, 2], bh[1], True),
        (W2[0], W2[1], W2[2], b2, False),
    ]

    h = x
    for w0, w1, w2, bl, act in layers:
        z1 = _sc_scatter(u, src, dst, zeros_c)
        partial, u = _tc_stage_a(z1, h, dis8, w0, w1, bl.reshape(1, _C))
        z2 = _sc_scatter(u, src, dst, zeros_c)
        stage_b = _tc_stage_b_act if act else _tc_stage_b_lin
        h, u = stage_b(z2, partial, h, dis8, w2)

    return _tc_pool(batch2, h, Wp1, bp1.reshape(1, 32),
                    Wp2, bp2.reshape(1, _G))


# speedup vs baseline: 8.9070x; 1.0694x over previous
import functools

import jax
import jax.numpy as jnp
from jax import lax
from jax.experimental import pallas as pl
from jax.experimental.pallas import tpu as pltpu
from jax.experimental.pallas import tpu_sc as plsc

_N = 10000
_E = 320000
_C = 128
_G = 16
_NC, _NS = 2, 16
_NW = _NC * _NS
_EPW = _E // _NW
_K = 50
_NCH = _EPW // _K
_K2 = 100
_NCH2 = _EPW // _K2
_PH = 2
_CPP = _NCH2 // _PH
_NP = 10112
_RPS = _NP // _NS
_DW = 16

_R = 1000
_GRID = _N // _R

_mesh = plsc.VectorSubcoreMesh(core_axis_name="c", subcore_axis_name="s",
                               num_cores=_NC, num_subcores=_NS)


@functools.partial(
    pl.kernel,
    out_type=jax.ShapeDtypeStruct((_NC, _NP, _C), jnp.float32),
    mesh=_mesh,
    scratch_types=[
        pltpu.VMEM((_CPP, _K2), jnp.int32),
        pltpu.VMEM((_CPP, _K2), jnp.int32),
        pltpu.VMEM((_K2, _C), jnp.float32),
        pltpu.VMEM((_K2, _C), jnp.float32),
        pltpu.VMEM_SHARED((_NP, _C), jnp.float32),
        pltpu.SemaphoreType.DMA,
        pltpu.SemaphoreType.DMA,
    ],
)
def _sc_scatter(u_hbm, gidx_hbm, sidx_hbm, zeros_hbm, out_hbm,
                gi_v, si_v, rows0_v, rows1_v, acc_sh, sem0, sem1):
    c = lax.axis_index("c")
    s = lax.axis_index("s")
    w = c * _NS + s
    pltpu.sync_copy(zeros_hbm, acc_sh.at[pl.ds(s * _RPS, _RPS)])
    plsc.subcore_barrier()

    def start(j, rows_v, sem):
        pltpu.make_async_copy(u_hbm.at[gi_v.at[j]], rows_v, sem).start()

    def drain(j, rows_v, sem):
        pltpu.make_async_copy(u_hbm.at[gi_v.at[j]], rows_v, sem).wait()
        pltpu.sync_copy(rows_v, acc_sh.at[si_v.at[j]], add=True)

    for p in range(_PH):
        pltpu.sync_copy(gidx_hbm.at[w, p], gi_v)
        pltpu.sync_copy(sidx_hbm.at[w, p], si_v)
        def chunk(j, carry):
            start(j, rows0_v, sem0)
            drain(j, rows0_v, sem0)
            return carry

        lax.fori_loop(0, _CPP, chunk, 0)

    plsc.subcore_barrier()
    pltpu.sync_copy(acc_sh.at[pl.ds(s * _RPS, _RPS)],
                    out_hbm.at[c, pl.ds(s * _RPS, _RPS)])


@functools.partial(
    pl.kernel,
    out_type=jax.ShapeDtypeStruct((_NC, _NP, _C), jnp.float32),
    mesh=_mesh,
    scratch_types=[
        pltpu.VMEM((_NCH, _K), jnp.int32),
        pltpu.VMEM((_K, _C), jnp.float32),
        pltpu.VMEM_SHARED((_NP, _C), jnp.float32),
    ],
)
def _sc_degree(src_hbm, ones_hbm, zeros_hbm, out_hbm, si_v, ones_v, acc_sh):
    c = lax.axis_index("c")
    s = lax.axis_index("s")
    w = c * _NS + s
    pltpu.sync_copy(zeros_hbm, acc_sh.at[pl.ds(s * _RPS, _RPS)])
    pltpu.sync_copy(src_hbm.at[w], si_v)
    pltpu.sync_copy(ones_hbm, ones_v)
    plsc.subcore_barrier()

    def chunk(j, carry):
        pltpu.sync_copy(ones_v, acc_sh.at[si_v.at[j]], add=True)
        return carry

    lax.fori_loop(0, _NCH, chunk, 0)
    plsc.subcore_barrier()
    pltpu.sync_copy(acc_sh.at[pl.ds(s * _RPS, _RPS)],
                    out_hbm.at[c, pl.ds(s * _RPS, _RPS)])


_HI = lax.Precision.HIGHEST


def _dot(a, b):
    return jnp.dot(a, b, preferred_element_type=jnp.float32, precision=_HI)


def _tc_prologue_body(degp_ref, x_ref, dis_ref, u0_ref):
    deg = degp_ref[0, :, 0:1] + degp_ref[1, :, 0:1]
    dis = jnp.where(deg > 0.0, lax.rsqrt(jnp.maximum(deg, 1e-30)), 0.0)
    dis_ref[...] = jnp.broadcast_to(dis, (_R, _DW))
    u0_ref[...] = dis * x_ref[...]


_tc_prologue = pl.pallas_call(
    _tc_prologue_body,
    grid=(_GRID,),
    in_specs=[
        pl.BlockSpec((_NC, _R, _C), lambda i: (0, i, 0)),
        pl.BlockSpec((_R, _C), lambda i: (i, 0)),
    ],
    out_specs=[
        pl.BlockSpec((_R, _DW), lambda i: (i, 0)),
        pl.BlockSpec((_R, _C), lambda i: (i, 0)),
    ],
    out_shape=[
        jax.ShapeDtypeStruct((_N, _DW), jnp.float32),
        jax.ShapeDtypeStruct((_N, _C), jnp.float32),
    ],
)


def _tc_stage_a_body(zp_ref, h_ref, dis_ref, w0_ref, w1_ref, b_ref,
                     part_ref, u1_ref):
    d = dis_ref[:, 0:1]
    z = zp_ref[0] + zp_ref[1]
    tx1 = -d * z
    part_ref[...] = _dot(h_ref[...], w0_ref[...]) + _dot(tx1, w1_ref[...]) \
        + b_ref[...]
    u1_ref[...] = d * tx1


_tc_stage_a = pl.pallas_call(
    _tc_stage_a_body,
    grid=(_GRID,),
    in_specs=[
        pl.BlockSpec((_NC, _R, _C), lambda i: (0, i, 0)),
        pl.BlockSpec((_R, _C), lambda i: (i, 0)),
        pl.BlockSpec((_R, _DW), lambda i: (i, 0)),
        pl.BlockSpec((_C, _C), lambda i: (0, 0)),
        pl.BlockSpec((_C, _C), lambda i: (0, 0)),
        pl.BlockSpec((1, _C), lambda i: (0, 0)),
    ],
    out_specs=[
        pl.BlockSpec((_R, _C), lambda i: (i, 0)),
        pl.BlockSpec((_R, _C), lambda i: (i, 0)),
    ],
    out_shape=[
        jax.ShapeDtypeStruct((_N, _C), jnp.float32),
        jax.ShapeDtypeStruct((_N, _C), jnp.float32),
    ],
)


def _tc_stage_b_body(zp_ref, part_ref, h_ref, dis_ref, w2_ref,
                     out_ref, un_ref, *, act):
    d = dis_ref[:, 0:1]
    z = zp_ref[0] + zp_ref[1]
    tx2 = -2.0 * d * z - h_ref[...]
    o = part_ref[...] + _dot(tx2, w2_ref[...])
    if act:
        o = o * lax.logistic(o)
    out_ref[...] = o
    un_ref[...] = d * o


def _make_stage_b(act):
    return pl.pallas_call(
        functools.partial(_tc_stage_b_body, act=act),
        grid=(_GRID,),
        in_specs=[
            pl.BlockSpec((_NC, _R, _C), lambda i: (0, i, 0)),
            pl.BlockSpec((_R, _C), lambda i: (i, 0)),
            pl.BlockSpec((_R, _C), lambda i: (i, 0)),
            pl.BlockSpec((_R, _DW), lambda i: (i, 0)),
            pl.BlockSpec((_C, _C), lambda i: (0, 0)),
        ],
        out_specs=[
            pl.BlockSpec((_R, _C), lambda i: (i, 0)),
            pl.BlockSpec((_R, _C), lambda i: (i, 0)),
        ],
        out_shape=[
            jax.ShapeDtypeStruct((_N, _C), jnp.float32),
            jax.ShapeDtypeStruct((_N, _C), jnp.float32),
        ],
    )


_tc_stage_b_act = _make_stage_b(True)
_tc_stage_b_lin = _make_stage_b(False)


def _tc_pool_body(b_ref, h_ref, wp1_ref, bp1_ref, wp2_ref, bp2_ref,
                  out_ref, acc_ref):
    i = pl.program_id(0)

    @pl.when(i == 0)
    def _zero():
        acc_ref[...] = jnp.zeros_like(acc_ref)

    onehot = (b_ref[...] == lax.broadcasted_iota(jnp.int32, (_R, _G), 1)
              ).astype(jnp.float32)
    acc_ref[...] += lax.dot_general(
        onehot, h_ref[...], (((0,), (0,)), ((), ())),
        preferred_element_type=jnp.float32, precision=_HI)

    @pl.when(i == pl.num_programs(0) - 1)
    def _head():
        p = acc_ref[...]
        o = jnp.maximum(_dot(p, wp1_ref[...]) + bp1_ref[...], 0.0)
        out_ref[...] = _dot(o, wp2_ref[...]) + bp2_ref[...]


_tc_pool = pl.pallas_call(
    _tc_pool_body,
    grid=(_GRID,),
    in_specs=[
        pl.BlockSpec((_R, 1), lambda i: (i, 0)),
        pl.BlockSpec((_R, _C), lambda i: (i, 0)),
        pl.BlockSpec((_C, 32), lambda i: (0, 0)),
        pl.BlockSpec((1, 32), lambda i: (0, 0)),
        pl.BlockSpec((32, _G), lambda i: (0, 0)),
        pl.BlockSpec((1, _G), lambda i: (0, 0)),
    ],
    out_specs=pl.BlockSpec((_G, _G), lambda i: (0, 0)),
    out_shape=jax.ShapeDtypeStruct((_G, _G), jnp.float32),
    scratch_shapes=[pltpu.VMEM((_G, _C), jnp.float32)],
)


def kernel(x, edge_index, batch, W1, b1, Wh, bh, W2, b2, Wp1, bp1, Wp2, bp2):
    src_d = edge_index[0].reshape(_NW, _NCH, _K)
    src = edge_index[0].reshape(_NW, _PH, _CPP, _K2)
    dst = edge_index[1].reshape(_NW, _PH, _CPP, _K2)
    zeros_c = jnp.zeros((_RPS, _C), jnp.float32)
    ones_d = jnp.ones((_K, _C), jnp.float32)
    batch2 = batch.reshape(_N, 1)

    degp = _sc_degree(src_d, ones_d, zeros_c)
    dis8, u = _tc_prologue(degp, x)

    layers = [
        (W1[0], W1[1], W1[2], b1, False),
        (Wh[0, 0], Wh[0, 1], Wh[0, 2], bh[0], True),
        (Wh[1, 0], Wh[1, 1], Wh[1, 2], bh[1], True),
        (W2[0], W2[1], W2[2], b2, False),
    ]

    h = x
    for w0, w1, w2, bl, act in layers:
        z1 = _sc_scatter(u, src, dst, zeros_c)
        partial, u = _tc_stage_a(z1, h, dis8, w0, w1, bl.reshape(1, _C))
        z2 = _sc_scatter(u, src, dst, zeros_c)
        stage_b = _tc_stage_b_act if act else _tc_stage_b_lin
        h, u = stage_b(z2, partial, h, dis8, w2)

    return _tc_pool(batch2, h, Wp1, bp1.reshape(1, 32),
                    Wp2, bp2.reshape(1, _G))
